# baseline (speedup 1.0000x reference)
import functools

import jax
import jax.numpy as jnp
from jax import lax
from jax.experimental import pallas as pl
from jax.experimental.pallas import tpu as pltpu
from jax.experimental.pallas import tpu_sc as plsc

N = 10000
E = 160000
NS = 16
K = 80
G = 125
NP = N
ESD = E // (2 * NS)
KD = 40
GD = ESD // KD
F32 = jnp.float32


def _make_prop(Dc, two):
    assert Dc % 128 == 0 and NP * Dc * 4 <= 8 * 1024 * 1024
    mesh = plsc.VectorSubcoreMesh(core_axis_name="c", subcore_axis_name="s")
    chunk_t = jax.ShapeDtypeStruct((N, Dc), F32)

    scratch = [
        pltpu.VMEM((2, K), jnp.int32),
        pltpu.VMEM((2, K), jnp.int32),
        pltpu.VMEM((K,), jnp.int32),
        pltpu.VMEM((K,), jnp.int32),
        pltpu.VMEM((K, Dc), F32),
        pltpu.VMEM((K, Dc), F32),
        pltpu.VMEM_SHARED((NP, Dc), F32),
        pltpu.SemaphoreType.DMA,
        pltpu.SemaphoreType.DMA,
        pltpu.SemaphoreType.DMA,
        pltpu.SemaphoreType.DMA,
        pltpu.SemaphoreType.DMA,
        pltpu.SemaphoreType.DMA,
    ]

    def run(g_hbm, eidx_hbm, out_hbm, s, sidx0, sidx1, sdst0, sdst1,
            gbuf0, gbuf1, acc, gsem0, gsem1, isem0, isem1, ssem0, ssem1):
        @pl.when(s < 15)
        def _():
            pltpu.sync_copy(g_hbm.at[pl.ds(s * 624, 624)],
                            acc.at[pl.ds(s * 624, 624)])

        @pl.when(s == 15)
        def _():
            pltpu.sync_copy(g_hbm.at[pl.ds(9360, 640)],
                            acc.at[pl.ds(9360, 640)])

        plsc.subcore_barrier()

        pltpu.sync_copy(eidx_hbm.at[s].at[0], sidx0)
        pltpu.async_copy(g_hbm.at[sidx0.at[0]], gbuf0, gsem0)
        pltpu.async_copy(eidx_hbm.at[s].at[1], sidx1, isem1)

        def group(i, _):
            nxt = i + 1
            nnxt = i + 2

            @pl.when(jnp.logical_and(nxt < G, nxt % 2 == 1))
            def _():
                pltpu.make_async_copy(eidx_hbm.at[s].at[nxt], sidx1,
                                      isem1).wait()

                @pl.when(i > 0)
                def _():
                    pltpu.make_async_copy(gbuf1, acc.at[sdst1], ssem1).wait()

                pltpu.async_copy(g_hbm.at[sidx1.at[0]], gbuf1, gsem1)

            @pl.when(jnp.logical_and(nxt < G, nxt % 2 == 0))
            def _():
                pltpu.make_async_copy(eidx_hbm.at[s].at[nxt], sidx0,
                                      isem0).wait()
                pltpu.make_async_copy(gbuf0, acc.at[sdst0], ssem0).wait()
                pltpu.async_copy(g_hbm.at[sidx0.at[0]], gbuf0, gsem0)

            @pl.when(i % 2 == 0)
            def _():
                pltpu.make_async_copy(g_hbm.at[sidx0.at[0]], gbuf0,
                                      gsem0).wait()
                for j in range(K // 16):
                    sdst0[pl.ds(j * 16, 16)] = sidx0[1, pl.ds(j * 16, 16)]
                pltpu.async_copy(gbuf0, acc.at[sdst0], ssem0, add=True)

            @pl.when(i % 2 == 1)
            def _():
                pltpu.make_async_copy(g_hbm.at[sidx1.at[0]], gbuf1,
                                      gsem1).wait()
                for j in range(K // 16):
                    sdst1[pl.ds(j * 16, 16)] = sidx1[1, pl.ds(j * 16, 16)]
                pltpu.async_copy(gbuf1, acc.at[sdst1], ssem1, add=True)

            @pl.when(jnp.logical_and(nnxt < G, nnxt % 2 == 0))
            def _():
                pltpu.async_copy(eidx_hbm.at[s].at[nnxt], sidx0, isem0)

            @pl.when(jnp.logical_and(nnxt < G, nnxt % 2 == 1))
            def _():
                pltpu.async_copy(eidx_hbm.at[s].at[nnxt], sidx1, isem1)

            return 0

        lax.fori_loop(0, G, group, 0)
        pltpu.make_async_copy(gbuf0, acc.at[sdst0], ssem0).wait()
        pltpu.make_async_copy(gbuf1, acc.at[sdst1], ssem1).wait()
        plsc.subcore_barrier()

        @pl.when(s < 15)
        def _():
            pltpu.sync_copy(acc.at[pl.ds(s * 624, 624)],
                            out_hbm.at[pl.ds(s * 624, 624)])

        @pl.when(s == 15)
        def _():
            pltpu.sync_copy(acc.at[pl.ds(9360, 640)],
                            out_hbm.at[pl.ds(9360, 640)])

    if two:
        @functools.partial(
            pl.kernel, out_type=[chunk_t, chunk_t], mesh=mesh,
            scratch_types=scratch)
        def prop(ga, gb, eidx_hbm, outa, outb,
                 sidx0, sidx1, sdst0, sdst1, gbuf0, gbuf1, acc,
                 gsem0, gsem1, isem0, isem1, ssem0, ssem1):
            c = lax.axis_index("c")
            s = lax.axis_index("s")

            @pl.when(c == 0)
            def _():
                run(ga, eidx_hbm, outa, s, sidx0, sidx1, sdst0, sdst1,
                    gbuf0, gbuf1, acc, gsem0, gsem1, isem0, isem1,
                    ssem0, ssem1)

            @pl.when(c == 1)
            def _():
                run(gb, eidx_hbm, outb, s, sidx0, sidx1, sdst0, sdst1,
                    gbuf0, gbuf1, acc, gsem0, gsem1, isem0, isem1,
                    ssem0, ssem1)
    else:
        @functools.partial(
            pl.kernel, out_type=chunk_t, mesh=mesh, scratch_types=scratch)
        def prop(ga, eidx_hbm, outa,
                 sidx0, sidx1, sdst0, sdst1, gbuf0, gbuf1, acc,
                 gsem0, gsem1, isem0, isem1, ssem0, ssem1):
            c = lax.axis_index("c")
            s = lax.axis_index("s")

            @pl.when(c == 0)
            def _():
                run(ga, eidx_hbm, outa, s, sidx0, sidx1, sdst0, sdst1,
                    gbuf0, gbuf1, acc, gsem0, gsem1, isem0, isem1,
                    ssem0, ssem1)

    return prop


_prop2 = _make_prop(128, two=True)


def _make_deg():
    mesh = plsc.VectorSubcoreMesh(core_axis_name="c", subcore_axis_name="s")
    out_t = jax.ShapeDtypeStruct((N, 16), F32)
    scratch = [
        pltpu.VMEM((GD, KD), jnp.int32),
        pltpu.VMEM((KD, 16), F32),
        pltpu.VMEM_SHARED((N, 16), F32),
        pltpu.SemaphoreType.DMA,
    ]

    @functools.partial(pl.kernel, out_type=[out_t, out_t], mesh=mesh,
                       scratch_types=scratch)
    def deg(ones_hbm, dst_hbm, out0, out1, edst, ones, acc, ssem):
        c = lax.axis_index("c")
        s = lax.axis_index("s")
        pltpu.sync_copy(dst_hbm.at[c].at[s], edst)
        pltpu.sync_copy(ones_hbm.at[pl.ds(0, KD)], ones)

        @pl.when(s < 15)
        def _():
            pltpu.sync_copy(ones_hbm.at[pl.ds(s * 624, 624)],
                            acc.at[pl.ds(s * 624, 624)])

        @pl.when(s == 15)
        def _():
            pltpu.sync_copy(ones_hbm.at[pl.ds(9360, 640)],
                            acc.at[pl.ds(9360, 640)])

        plsc.subcore_barrier()

        def chunk(b, _):
            for j in range(5):
                pltpu.async_copy(ones, acc.at[edst.at[b * 5 + j]], ssem,
                                 add=True)
            for j in range(5):
                pltpu.make_async_copy(ones, acc.at[edst.at[b * 5 + j]],
                                      ssem).wait()
            return 0

        lax.fori_loop(0, GD // 5, chunk, 0)
        plsc.subcore_barrier()

        @pl.when(jnp.logical_and(c == 0, s < 15))
        def _():
            pltpu.sync_copy(acc.at[pl.ds(s * 624, 624)],
                            out0.at[pl.ds(s * 624, 624)])

        @pl.when(jnp.logical_and(c == 0, s == 15))
        def _():
            pltpu.sync_copy(acc.at[pl.ds(9360, 640)],
                            out0.at[pl.ds(9360, 640)])

        @pl.when(jnp.logical_and(c == 1, s < 15))
        def _():
            pltpu.sync_copy(acc.at[pl.ds(s * 624, 624)],
                            out1.at[pl.ds(s * 624, 624)])

        @pl.when(jnp.logical_and(c == 1, s == 15))
        def _():
            pltpu.sync_copy(acc.at[pl.ds(9360, 640)],
                            out1.at[pl.ds(9360, 640)])

    return deg


_deg = _make_deg()


BMF = 200
BM = 400


def _dinv(d0_ref, d1_ref):
    return lax.rsqrt(d0_ref[...][:, :1] + d1_ref[...][:, :1] - 1.0)


def _feat_body(x_ref, d0_ref, d1_ref, wf1, bf1, wf2, bf2, wf3, bf3,
               o0, o1, o2, o3):
    xb = x_ref[...]
    f2 = jnp.maximum(jnp.dot(xb[:, :21], wf2[...],
                             preferred_element_type=F32) + bf2[...], 0.0)
    f1 = jnp.maximum(jnp.dot(xb[:, 21:6165], wf1[...],
                             preferred_element_type=F32) + bf1[...], 0.0)
    f3 = jnp.maximum(jnp.dot(xb[:, 6165:], wf3[...],
                             preferred_element_type=F32) + bf3[...], 0.0)
    feat = jnp.concatenate([f2, f1, f3, jnp.zeros((BMF, 43), F32)], axis=1)
    fd = feat * _dinv(d0_ref, d1_ref)
    for j, o in enumerate((o0, o1, o2, o3)):
        o[...] = fd[:, j * 128:(j + 1) * 128]


def _feat(x, d0, d1, wf1, bf1, wf2, bf2, wf3, bf3):
    full = lambda r, c: pl.BlockSpec((r, c), lambda i: (0, 0))
    chunk = lambda: pl.BlockSpec((BMF, 128), lambda i: (i, 0))
    return pl.pallas_call(
        _feat_body,
        grid=(N // BMF,),
        in_specs=[
            pl.BlockSpec((BMF, 6485), lambda i: (i, 0)),
            pl.BlockSpec((BMF, 16), lambda i: (i, 0)),
            pl.BlockSpec((BMF, 16), lambda i: (i, 0)),
            full(6144, 128), full(1, 128),
            full(21, 21), full(1, 21),
            full(320, 320), full(1, 320),
        ],
        out_specs=[chunk() for _ in range(4)],
        out_shape=[jax.ShapeDtypeStruct((N, 128), F32) for _ in range(4)],
    )(x, d0, d1, wf1, bf1, wf2, bf2, wf3, bf3)


def _mid1_body(s0, s1, s2, s3, d0_ref, d1_ref, wp1, bp1, wa1, ba1, *outs):
    dinv = _dinv(d0_ref, d1_ref)
    pf = jnp.concatenate([s0[...], s1[...], s2[...], s3[...]],
                         axis=1)[:, :469] * dinv
    xh = jnp.maximum(jnp.dot(pf, wp1[...], preferred_element_type=F32)
                     + bp1[...], 0.0)
    yh = jnp.maximum(jnp.dot(pf, wa1[...], preferred_element_type=F32)
                     + ba1[...], 0.0)
    g1 = jnp.concatenate([xh, yh, jnp.zeros((BM, 86), F32)], axis=1) * dinv
    for j, o in enumerate(outs):
        o[...] = g1[:, j * 128:(j + 1) * 128]


def _mid1(s0c, d0, d1, wp1, bp1, wa1, ba1):
    full = lambda r, c: pl.BlockSpec((r, c), lambda i: (0, 0))
    chunk = lambda: pl.BlockSpec((BM, 128), lambda i: (i, 0))
    return pl.pallas_call(
        _mid1_body,
        grid=(N // BM,),
        in_specs=[chunk() for _ in range(4)] + [
            pl.BlockSpec((BM, 16), lambda i: (i, 0)),
            pl.BlockSpec((BM, 16), lambda i: (i, 0)),
            full(469, 469), full(1, 469),
            full(469, 469), full(1, 469),
        ],
        out_specs=[chunk() for _ in range(8)],
        out_shape=[jax.ShapeDtypeStruct((N, 128), F32) for _ in range(8)],
    )(*s0c, d0, d1, wp1, bp1, wa1, ba1)


def _mid2_body(*refs):
    (s0, s1, s2, s3, s4, s5, s6, s7, d0_ref, d1_ref,
     wp2, bp2, wa2, ba2) = refs[:14]
    outs = refs[14:]
    dinv = _dinv(d0_ref, d1_ref)
    s1f = jnp.concatenate([r[...] for r in (s0, s1, s2, s3, s4, s5, s6, s7)],
                          axis=1)
    tx = s1f[:, :469] * dinv
    ty = s1f[:, 469:938] * dinv
    xh = jnp.maximum(jnp.dot(tx, wp2[...], preferred_element_type=F32)
                     + bp2[...], 0.0)
    yh = jnp.maximum(jnp.dot(ty, wa2[...], preferred_element_type=F32)
                     + ba2[...], 0.0)
    g2 = jnp.concatenate([xh, yh, jnp.zeros((BM, 172), F32)], axis=1) * dinv
    for j, o in enumerate(outs):
        o[...] = g2[:, j * 128:(j + 1) * 128]


def _mid2(s1c, d0, d1, wp2, bp2, wa2, ba2):
    full = lambda r, c: pl.BlockSpec((r, c), lambda i: (0, 0))
    chunk = lambda: pl.BlockSpec((BM, 128), lambda i: (i, 0))
    return pl.pallas_call(
        _mid2_body,
        grid=(N // BM,),
        in_specs=[chunk() for _ in range(8)] + [
            pl.BlockSpec((BM, 16), lambda i: (i, 0)),
            pl.BlockSpec((BM, 16), lambda i: (i, 0)),
            full(469, 938), full(1, 938),
            full(469, 938), full(1, 938),
        ],
        out_specs=[chunk() for _ in range(16)],
        out_shape=[jax.ShapeDtypeStruct((N, 128), F32) for _ in range(16)],
    )(*s1c, d0, d1, wp2, bp2, wa2, ba2)


def _final_body(*refs):
    s2c = refs[:16]
    (d0_ref, d1_ref, batch_ref, wp3, bp3, wg1, bg1, gam, bet,
     wg2, bg2, out_ref, sums, cnts) = refs[16:]
    i = pl.program_id(0)
    nblk = pl.num_programs(0)

    @pl.when(i == 0)
    def _():
        sums[...] = jnp.zeros_like(sums)
        cnts[...] = jnp.zeros_like(cnts)

    u = jnp.concatenate([r[...] for r in s2c],
                        axis=1)[:, :1876] * _dinv(d0_ref, d1_ref)
    z = jnp.maximum(jnp.dot(u, wp3[...], preferred_element_type=F32)
                    + bp3[...], 0.0)
    seg = batch_ref[0]
    oh = (lax.broadcasted_iota(jnp.int32, (32, BM), 0) == seg).astype(F32)
    sums[...] += jnp.dot(oh, z, preferred_element_type=F32)
    cnts[...] += jnp.sum(oh, axis=1, keepdims=True)

    @pl.when(i == nblk - 1)
    def _():
        pooled = sums[...] / jnp.maximum(cnts[...], 1.0)
        h = jnp.dot(pooled, wg1[...], preferred_element_type=F32) + bg1[...]
        mu = jnp.mean(h, axis=0, keepdims=True)
        var = jnp.mean((h - mu) ** 2, axis=0, keepdims=True)
        h = (h - mu) * lax.rsqrt(var + 1e-5) * gam[...] + bet[...]
        h = jnp.maximum(h, 0.0)
        o = jnp.dot(h, wg2[...], preferred_element_type=F32) + bg2[...]
        out_ref[...] = jax.nn.sigmoid(o)


def _final(s2c, d0, d1, batch3d, wp3, bp3, wg1, bg1, gam, bet, wg2, bg2):
    full = lambda r, c: pl.BlockSpec((r, c), lambda i: (0, 0))
    chunk = lambda: pl.BlockSpec((BM, 128), lambda i: (i, 0))
    return pl.pallas_call(
        _final_body,
        grid=(N // BM,),
        in_specs=[chunk() for _ in range(16)] + [
            pl.BlockSpec((BM, 16), lambda i: (i, 0)),
            pl.BlockSpec((BM, 16), lambda i: (i, 0)),
            pl.BlockSpec((1, 1, BM), lambda i: (i, 0, 0)),
            full(1876, 1876), full(1, 1876),
            full(1876, 1024), full(1, 1024),
            full(1, 1024), full(1, 1024),
            full(1024, 486), full(1, 486),
        ],
        out_specs=pl.BlockSpec((32, 486), lambda i: (0, 0)),
        out_shape=jax.ShapeDtypeStruct((32, 486), F32),
        scratch_shapes=[
            pltpu.VMEM((32, 1876), F32),
            pltpu.VMEM((32, 1), F32),
        ],
    )(*s2c, d0, d1, batch3d, wp3, bp3, wg1, bg1, gam, bet, wg2, bg2)


def kernel(x, edge_index, batch, W_f1, b_f1, W_f2, b_f2, W_f3, b_f3,
           W_p1, b_p1, W_p2, b_p2, W_a1, b_a1, W_a2, b_a2, W_p3, b_p3,
           W_g1, b_g1, gamma, beta, W_g2, b_g2):
    eidx = jnp.stack([edge_index[0].reshape(NS, G, K),
                      edge_index[1].reshape(NS, G, K)], axis=2)
    dst4 = edge_index[1].reshape(2, NS, GD, KD)
    ones = jnp.ones((N, 16), F32)
    row = lambda v: v.reshape(1, -1)

    d0, d1 = _deg(ones, dst4)

    def prop(chunks):
        nc = len(chunks) // 2
        parts = [_prop2(chunks[i], chunks[i + nc], eidx) for i in range(nc)]
        return [p[0] for p in parts] + [p[1] for p in parts]

    g0c = _feat(x, d0, d1, W_f1, row(b_f1), W_f2, row(b_f2), W_f3, row(b_f3))
    s0c = prop(list(g0c))
    g1c = _mid1(s0c, d0, d1, W_p1, row(b_p1), W_a1, row(b_a1))
    s1c = prop(list(g1c))
    g2c = _mid2(s1c, d0, d1, W_p2, row(b_p2), W_a2, row(b_a2))
    s2c = prop(list(g2c))

    out = _final(s2c, d0, d1, batch.reshape(N // BM, 1, BM), W_p3, row(b_p3),
                 W_g1, row(b_g1), row(gamma), row(beta), W_g2, row(b_g2))
    return out
